# baseline (speedup 1.0000x reference)
import functools

import jax
import jax.numpy as jnp
from jax import lax
from jax.experimental import pallas as pl
from jax.experimental.pallas import tpu as pltpu
from jax.experimental.pallas import tpu_sc as plsc

_N, _D = 1_000_000, 64
_NW = 32
_RPW = _N // _NW
_C = 250
_NCH = _RPW // _C


def _sc_body(x_hbm, o_hbm, in_buf, out_buf, in_sem, out_sem):
    wid = lax.axis_index("s") * 2 + lax.axis_index("c")
    base = wid * _RPW

    def in_copy(g, b):
        src = x_hbm.at[pl.ds(base + g * _C, _C), :]
        return pltpu.make_async_copy(src, in_buf.at[b], in_sem.at[b])

    def out_copy(g, b):
        dst = o_hbm.at[pl.ds(base + g * _C, _C), :]
        return pltpu.make_async_copy(out_buf.at[b], dst, out_sem.at[b])

    in_copy(0, 0).start()
    in_copy(1, 1).start()

    def chunk_body(g, _):
        b = lax.rem(g, 2)
        in_copy(g, b).wait()

        @pl.when(g >= 2)
        def _wait_prev_out():
            out_copy(g - 2, b).wait()

        def row_body(r, _):
            for k in range(4):
                v = in_buf[b, r, pl.ds(16 * k, 16)]
                out_buf[b, r, pl.ds(16 * k, 16)] = v + 1.0
            return ()

        lax.fori_loop(0, _C, row_body, (), unroll=2)

        @pl.when(jnp.logical_and(wid == 0, g == 0))
        def _fix_row1():
            for k in range(4):
                out_buf[0, 1, pl.ds(16 * k, 16)] = jnp.full(
                    (16,), 2.0, dtype=jnp.float32)

        out_copy(g, b).start()

        @pl.when(g + 2 < _NCH)
        def _start_next_in():
            in_copy(g + 2, b).start()

        return ()

    lax.fori_loop(0, _NCH, chunk_body, ())
    out_copy(_NCH - 2, lax.rem(_NCH - 2, 2)).wait()
    out_copy(_NCH - 1, lax.rem(_NCH - 1, 2)).wait()


@functools.partial(jax.jit, static_argnums=())
def _sc_add_one(x):
    mesh = plsc.VectorSubcoreMesh(core_axis_name="c", subcore_axis_name="s")
    return pl.kernel(
        _sc_body,
        out_type=jax.ShapeDtypeStruct((_N, _D), jnp.float32),
        mesh=mesh,
        scratch_types=[
            pltpu.VMEM((2, _C, _D), jnp.float32),
            pltpu.VMEM((2, _C, _D), jnp.float32),
            pltpu.SemaphoreType.DMA((2,)),
            pltpu.SemaphoreType.DMA((2,)),
        ],
        compiler_params=pltpu.CompilerParams(use_tc_tiling_on_sc=False),
    )(x)


def kernel(x):
    return _sc_add_one(x)


# speedup vs baseline: 1.3994x; 1.3994x over previous
import functools

import jax
import jax.numpy as jnp
from jax import lax
from jax.experimental import pallas as pl
from jax.experimental.pallas import tpu as pltpu
from jax.experimental.pallas import tpu_sc as plsc

_N, _D = 1_000_000, 64
_NW = 32
_RPW = 31_248
_C = 168
_NCH = _RPW // _C
_TAIL = _N - _NW * _RPW


def _sc_body(x_hbm, o_hbm, in_buf, out_buf, in_sem, out_sem):
    wid = lax.axis_index("s") * 2 + lax.axis_index("c")
    base = wid * _RPW

    def in_copy(g, b):
        src = x_hbm.at[pl.ds(base + g * _C, _C), :]
        return pltpu.make_async_copy(src, in_buf.at[b], in_sem.at[b])

    def out_copy(g, b):
        dst = o_hbm.at[pl.ds(base + g * _C, _C), :]
        return pltpu.make_async_copy(out_buf.at[b], dst, out_sem.at[b])

    in_copy(0, 0).start()
    in_copy(1, 1).start()

    def chunk_body(g, _):
        b = lax.rem(g, 2)
        in_copy(g, b).wait()

        @pl.when(g >= 2)
        def _wait_prev_out():
            out_copy(g - 2, b).wait()

        def row_body(r, _):
            for k in range(4):
                v = in_buf[b, r, pl.ds(16 * k, 16)]
                out_buf[b, r, pl.ds(16 * k, 16)] = v + 1.0
            return ()

        lax.fori_loop(0, _C, row_body, (), unroll=2)

        @pl.when(jnp.logical_and(wid == 0, g == 0))
        def _fix_row1():
            for k in range(4):
                out_buf[0, 1, pl.ds(16 * k, 16)] = jnp.full(
                    (16,), 2.0, dtype=jnp.float32)

        out_copy(g, b).start()

        @pl.when(g + 2 < _NCH)
        def _start_next_in():
            in_copy(g + 2, b).start()

        return ()

    lax.fori_loop(0, _NCH, chunk_body, ())
    out_copy(_NCH - 2, lax.rem(_NCH - 2, 2)).wait()
    out_copy(_NCH - 1, lax.rem(_NCH - 1, 2)).wait()

    @pl.when(wid == _NW - 1)
    def _tail():
        t0 = _NW * _RPW
        tin = pltpu.make_async_copy(
            x_hbm.at[pl.ds(t0, _TAIL), :], in_buf.at[0, pl.ds(0, _TAIL)],
            in_sem.at[0])
        tin.start()
        tin.wait()

        def trow(r, _):
            for k in range(4):
                v = in_buf[0, r, pl.ds(16 * k, 16)]
                out_buf[0, r, pl.ds(16 * k, 16)] = v + 1.0
            return ()

        lax.fori_loop(0, _TAIL, trow, ())
        tout = pltpu.make_async_copy(
            out_buf.at[0, pl.ds(0, _TAIL)], o_hbm.at[pl.ds(t0, _TAIL), :],
            out_sem.at[0])
        tout.start()
        tout.wait()


@jax.jit
def _sc_add_one(x):
    mesh = plsc.VectorSubcoreMesh(core_axis_name="c", subcore_axis_name="s")
    return pl.kernel(
        _sc_body,
        out_type=jax.ShapeDtypeStruct((_N, _D), jnp.float32),
        mesh=mesh,
        scratch_types=[
            pltpu.VMEM((2, _C, _D), jnp.float32),
            pltpu.VMEM((2, _C, _D), jnp.float32),
            pltpu.SemaphoreType.DMA((2,)),
            pltpu.SemaphoreType.DMA((2,)),
        ],
    )(x)


def kernel(x):
    return _sc_add_one(x)


# speedup vs baseline: 1.7891x; 1.2785x over previous
import jax
import jax.numpy as jnp
from jax.experimental import pallas as pl
from jax.experimental.pallas import tpu as pltpu

_ROWS_PER_BLOCK = 16000


def _body(x_ref, o_ref):
    o_ref[...] = x_ref[...] + 1.0

    @pl.when(pl.program_id(0) == 0)
    def _fix_row1():
        o_ref[1, :] = jnp.full((64,), 2.0, dtype=o_ref.dtype)


def kernel(x):
    n, d = x.shape
    rows = 20000
    grid = n // rows
    return pl.pallas_call(
        _body,
        grid=(grid,),
        in_specs=[pl.BlockSpec((rows, d), lambda i: (i, 0))],
        out_specs=pl.BlockSpec((rows, d), lambda i: (i, 0)),
        out_shape=jax.ShapeDtypeStruct((n, d), x.dtype),
        compiler_params=pltpu.CompilerParams(
            vmem_limit_bytes=100 * 1024 * 1024,
        ),
    )(x)
